# baseline (speedup 1.0000x reference)
import functools

import jax
import jax.numpy as jnp
from jax import lax
from jax.experimental import pallas as pl
from jax.experimental.pallas import tpu as pltpu
from jax.experimental.pallas import tpu_sc as plsc

NC = 2
NS = 16
LANES = 16
CHUNK = 128
BLK = 16


def _ceil_to(a, m):
  return (a + m - 1) // m * m


def _make_deg_kernel(e_pad, n_pad, d):
  ept = e_pad // (NC * NS)
  niter = ept // CHUNK
  rpt = n_pad // NS
  mesh = plsc.VectorSubcoreMesh(
      core_axis_name="c", subcore_axis_name="s", num_cores=NC, num_subcores=NS)

  @functools.partial(
      pl.kernel,
      out_type=jax.ShapeDtypeStruct((NC, n_pad, d), jnp.float32),
      mesh=mesh,
      scratch_types=[
          pltpu.VMEM((BLK, CHUNK), jnp.int32),
          pltpu.VMEM((CHUNK, d), jnp.float32),
          pltpu.VMEM((8, d), jnp.float32),
          pltpu.VMEM_SHARED((n_pad, d), jnp.float32),
      ],
  )
  def deg_kernel(ones_hbm, dst_hbm, out_hbm, didx, ones_v, zbuf, acc):
    c = lax.axis_index("c")
    s = lax.axis_index("s")
    wid = c * NS + s

    for i in range(8):
      for k in range(d // LANES):
        zbuf[i, pl.ds(k * LANES, LANES)] = jnp.zeros((LANES,), jnp.float32)
    pltpu.sync_copy(ones_hbm, ones_v)

    def zero_acc(r, _):
      pltpu.sync_copy(zbuf, acc.at[pl.ds(s * rpt + r * 8, 8)])
      return 0
    lax.fori_loop(0, rpt // 8, zero_acc, 0)
    plsc.subcore_barrier()

    def block(b, _):
      pltpu.sync_copy(dst_hbm.at[pl.ds(wid * niter + b * BLK, BLK)], didx)

      def step(k, _):
        pltpu.sync_copy(ones_v, acc.at[didx.at[k]], add=True)
        return 0
      lax.fori_loop(0, BLK, step, 0)
      return 0
    lax.fori_loop(0, niter // BLK, block, 0)

    plsc.subcore_barrier()
    pltpu.sync_copy(acc.at[pl.ds(s * rpt, rpt)],
                    out_hbm.at[c, pl.ds(s * rpt, rpt)])

  return deg_kernel


def _make_agg_kernel(e_pad, n, n_pad, d):
  nchunks = e_pad // CHUNK
  nblk0 = nchunks // (NS * BLK)
  rpt = n_pad // NS
  mesh = plsc.VectorSubcoreMesh(
      core_axis_name="c", subcore_axis_name="s", num_cores=NC, num_subcores=NS)

  @functools.partial(
      pl.kernel,
      out_type=jax.ShapeDtypeStruct((n_pad, d), jnp.float32),
      mesh=mesh,
      scratch_types=[
          pltpu.VMEM((BLK, CHUNK), jnp.int32),
          pltpu.VMEM((BLK, CHUNK), jnp.int32),
          pltpu.VMEM((CHUNK, d), jnp.float32),
          pltpu.VMEM((CHUNK, d), jnp.float32),
          pltpu.VMEM((8, d), jnp.float32),
          pltpu.VMEM_SHARED((n_pad, d), jnp.float32),
          pltpu.SemaphoreType.DMA,
          pltpu.SemaphoreType.DMA,
      ],
  )
  def agg_kernel(tbl_hbm, src_hbm, dst_hbm, out_hbm,
                 sidx, didx, rows0, rows1, zbuf, acc, sem0, sem1):
    c = lax.axis_index("c")
    s = lax.axis_index("s")

    @pl.when(c == 0)
    def _():
      cbase = s * (nblk0 * BLK)

      for i in range(8):
        for k in range(d // LANES):
          zbuf[i, pl.ds(k * LANES, LANES)] = jnp.zeros((LANES,), jnp.float32)

      def zero_acc(r, _):
        pltpu.sync_copy(zbuf, acc.at[pl.ds(s * rpt + r * 8, 8)])
        return 0
      lax.fori_loop(0, rpt // 8, zero_acc, 0)
      plsc.subcore_barrier()

      def gather_start(k, rows, sem):
        pltpu.async_copy(tbl_hbm.at[sidx.at[k]], rows, sem)

      def gather_wait(k, rows, sem):
        pltpu.make_async_copy(tbl_hbm.at[sidx.at[k]], rows, sem).wait()

      def scatter(k, rows):
        pltpu.sync_copy(rows, acc.at[didx.at[k]], add=True)

      def block(b, _):
        row0 = cbase + b * BLK
        pltpu.sync_copy(src_hbm.at[pl.ds(row0, BLK)], sidx)
        pltpu.sync_copy(dst_hbm.at[pl.ds(row0, BLK)], didx)
        gather_start(0, rows0, sem0)

        def pair(t, _):
          k = 2 * t
          gather_start(k + 1, rows1, sem1)
          gather_wait(k, rows0, sem0)
          scatter(k, rows0)

          @pl.when(t < BLK // 2 - 1)
          def _():
            gather_start(k + 2, rows0, sem0)

          gather_wait(k + 1, rows1, sem1)
          scatter(k + 1, rows1)
          return 0
        lax.fori_loop(0, BLK // 2, pair, 0)
        return 0
      lax.fori_loop(0, nblk0, block, 0)

      plsc.subcore_barrier()
      pltpu.sync_copy(acc.at[pl.ds(s * rpt, rpt)],
                      out_hbm.at[pl.ds(s * rpt, rpt)])

  return agg_kernel


def _tc_matmul(x, w1, n, d):
  def body(x_ref, w_ref, h_ref):
    h_ref[...] = jnp.dot(x_ref[...], w_ref[...],
                         preferred_element_type=jnp.float32)

  return pl.pallas_call(
      body,
      out_shape=jax.ShapeDtypeStruct((n, d), jnp.float32),
  )(x, w1)


def _tc_first(h1, degp, n, d):
  def body(h_ref, degp_ref, h1p_ref, dinv_ref):
    cnt = degp_ref[0, 0:n, 0:1] + degp_ref[1, 0:n, 0:1]
    dinv = lax.rsqrt(cnt + 1.0)
    dinvb = lax.broadcast_in_dim(dinv, (n, d), (0, 1))
    h1p_ref[...] = h_ref[...] * dinvb
    dinv_ref[...] = dinvb

  return pl.pallas_call(
      body,
      out_shape=(jax.ShapeDtypeStruct((n, d), jnp.float32),
                 jax.ShapeDtypeStruct((n, d), jnp.float32)),
  )(h1, degp)


def _tc_mid(parts, h1p, dinvb, b1, w2, n, d):
  def body(p_ref, h1p_ref, dinv_ref, b1_ref, w2_ref, h2p_ref):
    agg = p_ref[0:n, :] + h1p_ref[...]
    z = jnp.maximum(dinv_ref[...] * agg + b1_ref[...][None, :], 0.0)
    h2 = jnp.dot(z, w2_ref[...], preferred_element_type=jnp.float32)
    h2p_ref[...] = h2 * dinv_ref[...]

  return pl.pallas_call(
      body,
      out_shape=jax.ShapeDtypeStruct((n, d), jnp.float32),
  )(parts, h1p, dinvb, b1, w2)


def _tc_last(parts, h2p, dinvb, b2, n, d):
  def body(p_ref, h2p_ref, dinv_ref, b2_ref, out_ref):
    agg = p_ref[0:n, :] + h2p_ref[...]
    out_ref[...] = dinv_ref[...] * agg + b2_ref[...][None, :]

  return pl.pallas_call(
      body,
      out_shape=jax.ShapeDtypeStruct((n, d), jnp.float32),
  )(parts, h2p, dinvb, b2)


def kernel(x, edge_index, W1, b1, W2, b2):
  n, d = x.shape
  e = edge_index.shape[1]
  e_pad = _ceil_to(e, NC * NS * CHUNK * BLK)
  n_pad = _ceil_to(n + 1, NS * 8)

  src = edge_index[0].astype(jnp.int32)
  dst = edge_index[1].astype(jnp.int32)
  pad = e_pad - e
  if pad:
    src = jnp.concatenate([src, jnp.zeros((pad,), jnp.int32)])
    dst = jnp.concatenate([dst, jnp.full((pad,), n, jnp.int32)])
  src = src.reshape(e_pad // CHUNK, CHUNK)
  dst = dst.reshape(e_pad // CHUNK, CHUNK)

  deg_kernel = _make_deg_kernel(e_pad, n_pad, d)
  agg_kernel = _make_agg_kernel(e_pad, n, n_pad, d)

  ones_blk = jnp.ones((CHUNK, d), jnp.float32)
  degp = deg_kernel(ones_blk, dst)
  h1 = _tc_matmul(x, W1, n, d)
  h1p, dinvb = _tc_first(h1, degp, n, d)
  p1 = agg_kernel(h1p, src, dst)
  h2p = _tc_mid(p1, h1p, dinvb, b1, W2, n, d)
  p2 = agg_kernel(h2p, src, dst)
  return _tc_last(p2, h2p, dinvb, b2, n, d)


# speedup vs baseline: 3.6086x; 3.6086x over previous
import functools

import jax
import jax.numpy as jnp
from jax import lax
from jax.experimental import pallas as pl
from jax.experimental.pallas import tpu as pltpu
from jax.experimental.pallas import tpu_sc as plsc

NC = 2
NS = 16
LANES = 16
CHUNK = 128
BLK = 16


def _ceil_to(a, m):
  return (a + m - 1) // m * m


def _make_deg_kernel(e_pad, n_pad, d):
  ept = e_pad // (NC * NS)
  niter = ept // CHUNK
  rpt = n_pad // NS
  mesh = plsc.VectorSubcoreMesh(
      core_axis_name="c", subcore_axis_name="s", num_cores=NC, num_subcores=NS)

  @functools.partial(
      pl.kernel,
      out_type=jax.ShapeDtypeStruct((NC, n_pad, d), jnp.float32),
      mesh=mesh,
      scratch_types=[
          pltpu.VMEM((BLK, CHUNK), jnp.int32),
          pltpu.VMEM((CHUNK, d), jnp.float32),
          pltpu.VMEM((8, d), jnp.float32),
          pltpu.VMEM_SHARED((n_pad, d), jnp.float32),
      ],
  )
  def deg_kernel(ones_hbm, dst_hbm, out_hbm, didx, ones_v, zbuf, acc):
    c = lax.axis_index("c")
    s = lax.axis_index("s")
    wid = c * NS + s

    for i in range(8):
      for k in range(d // LANES):
        zbuf[i, pl.ds(k * LANES, LANES)] = jnp.zeros((LANES,), jnp.float32)
    pltpu.sync_copy(ones_hbm, ones_v)

    def zero_acc(r, _):
      pltpu.sync_copy(zbuf, acc.at[pl.ds(s * rpt + r * 8, 8)])
      return 0
    lax.fori_loop(0, rpt // 8, zero_acc, 0)
    plsc.subcore_barrier()

    def block(b, _):
      pltpu.sync_copy(dst_hbm.at[pl.ds(wid * niter + b * BLK, BLK)], didx)

      def step(k, _):
        pltpu.sync_copy(ones_v, acc.at[didx.at[k]], add=True)
        return 0
      lax.fori_loop(0, BLK, step, 0)
      return 0
    lax.fori_loop(0, niter // BLK, block, 0)

    plsc.subcore_barrier()
    pltpu.sync_copy(acc.at[pl.ds(s * rpt, rpt)],
                    out_hbm.at[c, pl.ds(s * rpt, rpt)])

  return deg_kernel


def _make_agg_kernel(e_pad, n, n_pad, d):
  nchunks = e_pad // CHUNK
  cpc = nchunks // NC
  nblk = cpc // (NS * BLK)
  rpt = n_pad // NS
  mesh = plsc.VectorSubcoreMesh(
      core_axis_name="c", subcore_axis_name="s", num_cores=NC, num_subcores=NS)

  @functools.partial(
      pl.kernel,
      out_type=jax.ShapeDtypeStruct((NC, n_pad, d), jnp.float32),
      mesh=mesh,
      scratch_types=[
          pltpu.VMEM((BLK, CHUNK), jnp.int32),
          pltpu.VMEM((BLK, CHUNK), jnp.int32),
          pltpu.VMEM((CHUNK, d), jnp.float32),
          pltpu.VMEM((CHUNK, d), jnp.float32),
          pltpu.VMEM((8, d), jnp.float32),
          pltpu.VMEM_SHARED((n_pad, d), jnp.float32),
          pltpu.SemaphoreType.DMA,
          pltpu.SemaphoreType.DMA,
      ],
  )
  def agg_kernel(tbl_hbm, src_hbm, dst_hbm, out_hbm,
                 sidx, didx, rows0, rows1, zbuf, acc, sem0, sem1):
    c = lax.axis_index("c")
    s = lax.axis_index("s")
    cbase = c * cpc + s * (nblk * BLK)

    for i in range(8):
      for k in range(d // LANES):
        zbuf[i, pl.ds(k * LANES, LANES)] = jnp.zeros((LANES,), jnp.float32)

    def zero_acc(r, _):
      pltpu.sync_copy(zbuf, acc.at[pl.ds(s * rpt + r * 8, 8)])
      return 0
    lax.fori_loop(0, rpt // 8, zero_acc, 0)
    plsc.subcore_barrier()

    def gather_start(k, rows, sem):
      pltpu.async_copy(tbl_hbm.at[sidx.at[k]], rows, sem)

    def gather_wait(k, rows, sem):
      pltpu.make_async_copy(tbl_hbm.at[sidx.at[k]], rows, sem).wait()

    def scatter(k, rows):
      pltpu.sync_copy(rows, acc.at[didx.at[k]], add=True)

    def block(b, _):
      row0 = cbase + b * BLK
      pltpu.sync_copy(src_hbm.at[pl.ds(row0, BLK)], sidx)
      pltpu.sync_copy(dst_hbm.at[pl.ds(row0, BLK)], didx)
      gather_start(0, rows0, sem0)

      def pair(t, _):
        k = 2 * t
        gather_start(k + 1, rows1, sem1)
        gather_wait(k, rows0, sem0)
        scatter(k, rows0)

        @pl.when(t < BLK // 2 - 1)
        def _():
          gather_start(k + 2, rows0, sem0)

        gather_wait(k + 1, rows1, sem1)
        scatter(k + 1, rows1)
        return 0
      lax.fori_loop(0, BLK // 2, pair, 0)
      return 0
    lax.fori_loop(0, nblk, block, 0)

    plsc.subcore_barrier()
    pltpu.sync_copy(acc.at[pl.ds(s * rpt, rpt)],
                    out_hbm.at[c, pl.ds(s * rpt, rpt)])

  return agg_kernel


def _tc_matmul(x, w1, n, d):
  def body(x_ref, w_ref, h_ref):
    h_ref[...] = jnp.dot(x_ref[...], w_ref[...],
                         preferred_element_type=jnp.float32)

  return pl.pallas_call(
      body,
      out_shape=jax.ShapeDtypeStruct((n, d), jnp.float32),
  )(x, w1)


def _tc_first(h1, degp, n, d):
  def body(h_ref, degp_ref, h1p_ref, dinv_ref):
    cnt = degp_ref[0, 0:n, 0:1] + degp_ref[1, 0:n, 0:1]
    dinv = lax.rsqrt(cnt + 1.0)
    dinvb = lax.broadcast_in_dim(dinv, (n, d), (0, 1))
    h1p_ref[...] = h_ref[...] * dinvb
    dinv_ref[...] = dinvb

  return pl.pallas_call(
      body,
      out_shape=(jax.ShapeDtypeStruct((n, d), jnp.float32),
                 jax.ShapeDtypeStruct((n, d), jnp.float32)),
  )(h1, degp)


def _tc_mid(parts, h1p, dinvb, b1, w2, n, d):
  def body(p_ref, h1p_ref, dinv_ref, b1_ref, w2_ref, h2p_ref):
    agg = p_ref[0, 0:n, :] + p_ref[1, 0:n, :] + h1p_ref[...]
    z = jnp.maximum(dinv_ref[...] * agg + b1_ref[...][None, :], 0.0)
    h2 = jnp.dot(z, w2_ref[...], preferred_element_type=jnp.float32)
    h2p_ref[...] = h2 * dinv_ref[...]

  return pl.pallas_call(
      body,
      out_shape=jax.ShapeDtypeStruct((n, d), jnp.float32),
  )(parts, h1p, dinvb, b1, w2)


def _tc_last(parts, h2p, dinvb, b2, n, d):
  def body(p_ref, h2p_ref, dinv_ref, b2_ref, out_ref):
    agg = p_ref[0, 0:n, :] + p_ref[1, 0:n, :] + h2p_ref[...]
    out_ref[...] = dinv_ref[...] * agg + b2_ref[...][None, :]

  return pl.pallas_call(
      body,
      out_shape=jax.ShapeDtypeStruct((n, d), jnp.float32),
  )(parts, h2p, dinvb, b2)


def kernel(x, edge_index, W1, b1, W2, b2):
  n, d = x.shape
  e = edge_index.shape[1]
  e_pad = _ceil_to(e, NC * NS * CHUNK * BLK)
  n_pad = _ceil_to(n + 1, NS * 8)

  src = edge_index[0].astype(jnp.int32)
  dst = edge_index[1].astype(jnp.int32)
  pad = e_pad - e
  if pad:
    src = jnp.concatenate([src, jnp.arange(pad, dtype=jnp.int32) % n])
    dst = jnp.concatenate([dst, jnp.full((pad,), n, jnp.int32)])
  src = src.reshape(e_pad // CHUNK, CHUNK)
  dst = dst.reshape(e_pad // CHUNK, CHUNK)

  deg_kernel = _make_deg_kernel(e_pad, n_pad, d)
  agg_kernel = _make_agg_kernel(e_pad, n, n_pad, d)

  ones_blk = jnp.ones((CHUNK, d), jnp.float32)
  degp = deg_kernel(ones_blk, dst)
  h1 = _tc_matmul(x, W1, n, d)
  h1p, dinvb = _tc_first(h1, degp, n, d)
  p1 = agg_kernel(h1p, src, dst)
  h2p = _tc_mid(p1, h1p, dinvb, b1, W2, n, d)
  p2 = agg_kernel(h2p, src, dst)
  return _tc_last(p2, h2p, dinvb, b2, n, d)


# speedup vs baseline: 3.6894x; 1.0224x over previous
import functools

import jax
import jax.numpy as jnp
from jax import lax
from jax.experimental import pallas as pl
from jax.experimental.pallas import tpu as pltpu
from jax.experimental.pallas import tpu_sc as plsc

NC = 2
NS = 16
LANES = 16
CHUNK = 128
BLK = 16


def _ceil_to(a, m):
  return (a + m - 1) // m * m


def _make_deg_kernel(e_pad, n_pad, d):
  ept = e_pad // (NC * NS)
  niter = ept // CHUNK
  rpt = n_pad // NS
  mesh = plsc.VectorSubcoreMesh(
      core_axis_name="c", subcore_axis_name="s", num_cores=NC, num_subcores=NS)

  @functools.partial(
      pl.kernel,
      out_type=jax.ShapeDtypeStruct((NC, n_pad, d), jnp.float32),
      mesh=mesh,
      scratch_types=[
          pltpu.VMEM((BLK, CHUNK), jnp.int32),
          pltpu.VMEM((CHUNK, d), jnp.float32),
          pltpu.VMEM((8, d), jnp.float32),
          pltpu.VMEM_SHARED((n_pad, d), jnp.float32),
      ],
  )
  def deg_kernel(ones_hbm, ei_hbm, out_hbm, didx, ones_v, zbuf, acc):
    c = lax.axis_index("c")
    s = lax.axis_index("s")
    wid = c * NS + s

    for i in range(8):
      for k in range(d // LANES):
        zbuf[i, pl.ds(k * LANES, LANES)] = jnp.zeros((LANES,), jnp.float32)
    pltpu.sync_copy(ones_hbm, ones_v)

    def zero_acc(r, _):
      pltpu.sync_copy(zbuf, acc.at[pl.ds(s * rpt + r * 8, 8)])
      return 0
    lax.fori_loop(0, rpt // 8, zero_acc, 0)
    plsc.subcore_barrier()

    def block(b, _):
      pltpu.sync_copy(ei_hbm.at[1, pl.ds(wid * niter + b * BLK, BLK)], didx)

      def step(k, _):
        pltpu.sync_copy(ones_v, acc.at[didx.at[k]], add=True)
        return 0
      lax.fori_loop(0, BLK, step, 0)
      return 0
    lax.fori_loop(0, niter // BLK, block, 0)

    plsc.subcore_barrier()
    pltpu.sync_copy(acc.at[pl.ds(s * rpt, rpt)],
                    out_hbm.at[c, pl.ds(s * rpt, rpt)])

  return deg_kernel


def _make_agg_kernel(e_pad, n, n_pad, d):
  nchunks = e_pad // CHUNK
  cpc = nchunks // NC
  nblk = cpc // (NS * BLK)
  rpt = n_pad // NS
  mesh = plsc.VectorSubcoreMesh(
      core_axis_name="c", subcore_axis_name="s", num_cores=NC, num_subcores=NS)

  @functools.partial(
      pl.kernel,
      out_type=jax.ShapeDtypeStruct((NC, n_pad, d), jnp.float32),
      mesh=mesh,
      scratch_types=[
          pltpu.VMEM((BLK, CHUNK), jnp.int32),
          pltpu.VMEM((BLK, CHUNK), jnp.int32),
          pltpu.VMEM((CHUNK, d), jnp.float32),
          pltpu.VMEM((CHUNK, d), jnp.float32),
          pltpu.VMEM((8, d), jnp.float32),
          pltpu.VMEM_SHARED((n_pad, d), jnp.float32),
          pltpu.SemaphoreType.DMA,
          pltpu.SemaphoreType.DMA,
      ],
  )
  def agg_kernel(tbl_hbm, ei_hbm, out_hbm,
                 sidx, didx, rows0, rows1, zbuf, acc, sem0, sem1):
    c = lax.axis_index("c")
    s = lax.axis_index("s")
    cbase = c * cpc + s * (nblk * BLK)

    for i in range(8):
      for k in range(d // LANES):
        zbuf[i, pl.ds(k * LANES, LANES)] = jnp.zeros((LANES,), jnp.float32)

    def zero_acc(r, _):
      pltpu.sync_copy(zbuf, acc.at[pl.ds(s * rpt + r * 8, 8)])
      return 0
    lax.fori_loop(0, rpt // 8, zero_acc, 0)
    plsc.subcore_barrier()

    def gather_start(k, rows, sem):
      pltpu.async_copy(tbl_hbm.at[sidx.at[k]], rows, sem)

    def gather_wait(k, rows, sem):
      pltpu.make_async_copy(tbl_hbm.at[sidx.at[k]], rows, sem).wait()

    def scatter(k, rows):
      pltpu.sync_copy(rows, acc.at[didx.at[k]], add=True)

    def block(b, _):
      row0 = cbase + b * BLK
      pltpu.sync_copy(ei_hbm.at[0, pl.ds(row0, BLK)], sidx)
      pltpu.sync_copy(ei_hbm.at[1, pl.ds(row0, BLK)], didx)
      gather_start(0, rows0, sem0)

      def pair(t, _):
        k = 2 * t
        gather_start(k + 1, rows1, sem1)
        gather_wait(k, rows0, sem0)
        scatter(k, rows0)

        @pl.when(t < BLK // 2 - 1)
        def _():
          gather_start(k + 2, rows0, sem0)

        gather_wait(k + 1, rows1, sem1)
        scatter(k + 1, rows1)
        return 0
      lax.fori_loop(0, BLK // 2, pair, 0)
      return 0
    lax.fori_loop(0, nblk, block, 0)

    plsc.subcore_barrier()
    pltpu.sync_copy(acc.at[pl.ds(s * rpt, rpt)],
                    out_hbm.at[c, pl.ds(s * rpt, rpt)])

  return agg_kernel


def _tc_matmul(x, w1, n, d):
  def body(x_ref, w_ref, h_ref):
    h_ref[...] = jnp.dot(x_ref[...], w_ref[...],
                         preferred_element_type=jnp.float32)

  return pl.pallas_call(
      body,
      out_shape=jax.ShapeDtypeStruct((n, d), jnp.float32),
  )(x, w1)


def _tc_first(h1, degp, n, d):
  def body(h_ref, degp_ref, h1p_ref, dinv_ref):
    cnt = degp_ref[0, 0:n, 0:1] + degp_ref[1, 0:n, 0:1]
    dinv = lax.rsqrt(cnt + 1.0)
    dinvb = lax.broadcast_in_dim(dinv, (n, d), (0, 1))
    h1p_ref[...] = h_ref[...] * dinvb
    dinv_ref[...] = dinvb

  return pl.pallas_call(
      body,
      out_shape=(jax.ShapeDtypeStruct((n, d), jnp.float32),
                 jax.ShapeDtypeStruct((n, d), jnp.float32)),
  )(h1, degp)


def _tc_mid(parts, h1p, dinvb, b1, w2, n, d):
  def body(p_ref, h1p_ref, dinv_ref, b1_ref, w2_ref, h2p_ref):
    agg = p_ref[0, 0:n, :] + p_ref[1, 0:n, :] + h1p_ref[...]
    z = jnp.maximum(dinv_ref[...] * agg + b1_ref[...][None, :], 0.0)
    h2 = jnp.dot(z, w2_ref[...], preferred_element_type=jnp.float32)
    h2p_ref[...] = h2 * dinv_ref[...]

  return pl.pallas_call(
      body,
      out_shape=jax.ShapeDtypeStruct((n, d), jnp.float32),
  )(parts, h1p, dinvb, b1, w2)


def _tc_last(parts, h2p, dinvb, b2, n, d):
  def body(p_ref, h2p_ref, dinv_ref, b2_ref, out_ref):
    agg = p_ref[0, 0:n, :] + p_ref[1, 0:n, :] + h2p_ref[...]
    out_ref[...] = dinv_ref[...] * agg + b2_ref[...][None, :]

  return pl.pallas_call(
      body,
      out_shape=jax.ShapeDtypeStruct((n, d), jnp.float32),
  )(parts, h2p, dinvb, b2)


def kernel(x, edge_index, W1, b1, W2, b2):
  n, d = x.shape
  e = edge_index.shape[1]
  e_pad = _ceil_to(e, NC * NS * CHUNK * BLK)
  n_pad = _ceil_to(n + 1, NS * 8)

  ei = edge_index.astype(jnp.int32)
  pad = e_pad - e
  if pad:
    pads = jnp.stack([jnp.arange(pad, dtype=jnp.int32) % n,
                      jnp.full((pad,), n, jnp.int32)])
    ei = jnp.concatenate([ei, pads], axis=1)
  ei = ei.reshape(2, e_pad // CHUNK, CHUNK)

  deg_kernel = _make_deg_kernel(e_pad, n_pad, d)
  agg_kernel = _make_agg_kernel(e_pad, n, n_pad, d)

  ones_blk = jnp.ones((CHUNK, d), jnp.float32)
  degp = deg_kernel(ones_blk, ei)
  h1 = _tc_matmul(x, W1, n, d)
  h1p, dinvb = _tc_first(h1, degp, n, d)
  p1 = agg_kernel(h1p, ei)
  h2p = _tc_mid(p1, h1p, dinvb, b1, W2, n, d)
  p2 = agg_kernel(h2p, ei)
  return _tc_last(p2, h2p, dinvb, b2, n, d)
